# baseline (speedup 1.0000x reference)
import jax
import jax.numpy as jnp
from jax.experimental import pallas as pl
from jax.experimental.pallas import tpu as pltpu

_BLOCK = 296


def kernel(patch, W):
    n = patch.shape[1] + 1
    d = W.shape[1]
    steps = (n + _BLOCK - 1) // _BLOCK

    def body(w_ref, o_ref):
        o_ref[0, ...] = w_ref[...]

    out = pl.pallas_call(
        body,
        out_shape=jax.ShapeDtypeStruct((1, n, d), W.dtype),
        grid=(steps,),
        in_specs=[pl.BlockSpec((_BLOCK, d), lambda i: (i, 0))],
        out_specs=pl.BlockSpec((1, _BLOCK, d), lambda i: (0, i, 0)),
        compiler_params=pltpu.CompilerParams(
            dimension_semantics=("parallel",),
            skip_device_barrier=True,
            disable_bounds_checks=True,
            disable_semaphore_checks=True,
        ),
    )(W)
    return out
